# baseline (speedup 1.0000x reference)
import functools

import jax
import jax.numpy as jnp
from jax import lax
from jax.experimental import pallas as pl
from jax.experimental.pallas import tpu as pltpu
from jax.experimental.pallas import tpu_sc as plsc

_MAX_DIST = 1000
_NUM_CORES = 2
_NUM_SUBCORES = 16


@functools.lru_cache(maxsize=None)
def _build_sc_kernel(S, H, VLEN):
    NW = _NUM_CORES * _NUM_SUBCORES
    ROWS = H * S
    RPW = ROWS // NW
    K = 16
    NG = RPW // K
    CH = 16
    UNROLL = 8
    assert ROWS % NW == 0 and S % RPW == 0 and RPW % (2 * K) == 0
    assert S % (CH * UNROLL) == 0

    mesh = plsc.VectorSubcoreMesh(
        core_axis_name="c", subcore_axis_name="s",
        num_cores=_NUM_CORES, num_subcores=_NUM_SUBCORES)

    @functools.partial(
        pl.kernel,
        out_type=jax.ShapeDtypeStruct((ROWS * S,), jnp.float32),
        mesh=mesh,
        scratch_types=[
            pltpu.VMEM((VLEN,), jnp.float32),
            pltpu.VMEM((K * S,), jnp.float32),
            pltpu.VMEM((K * S,), jnp.float32),
            pltpu.SemaphoreType.DMA,
            pltpu.SemaphoreType.DMA,
        ],
    )
    def sc_kernel(v_hbm, out_hbm, vsrc, buf0, buf1, sem0, sem1):
        wid = lax.axis_index("s") * _NUM_CORES + lax.axis_index("c")
        r0 = wid * RPW
        h = r0 // S
        i0 = r0 - h * S

        pltpu.sync_copy(v_hbm.at[h], vsrc)

        def fill(buf, g):
            for d in range(K):
                start = (S - 1) - (i0 + g * K + d)
                base = d * S

                def body(c, _, start=start, base=base):
                    off = c * (CH * UNROLL)
                    for u in range(UNROLL):
                        o = off + u * CH
                        buf[pl.ds(base + o, CH)] = vsrc[pl.ds(start + o, CH)]
                    return _
                lax.fori_loop(0, S // (CH * UNROLL), body, None)

        def send(buf, sem, g):
            pltpu.make_async_copy(
                buf, out_hbm.at[pl.ds((r0 + g * K) * S, K * S)], sem).start()

        def wait(sem):
            pltpu.make_async_copy(
                buf0, out_hbm.at[pl.ds(r0 * S, K * S)], sem).wait()

        fill(buf0, 0)
        send(buf0, sem0, 0)
        fill(buf1, 1)
        send(buf1, sem1, 1)

        def pair(gg, _):
            g0 = 2 * gg
            wait(sem0)
            fill(buf0, g0)
            send(buf0, sem0, g0)
            wait(sem1)
            fill(buf1, g0 + 1)
            send(buf1, sem1, g0 + 1)
            return _
        lax.fori_loop(1, NG // 2, pair, None)
        wait(sem0)
        wait(sem1)

    return sc_kernel


def kernel(x, bias_values):
    S = x.shape[1]
    NB, H = bias_values.shape
    assert NB == 2 * _MAX_DIST
    VLEN = 2 * S + 16
    n_left = (S - 1) - _MAX_DIST
    n_right = VLEN - n_left - NB
    assert n_left >= 0 and n_right >= 1

    v = jnp.concatenate([
        jnp.broadcast_to(bias_values[0], (n_left, H)),
        bias_values,
        jnp.broadcast_to(bias_values[-1], (n_right, H)),
    ], axis=0).T

    out = _build_sc_kernel(S, H, VLEN)(v)
    return out.reshape(1, H, S, S)


# speedup vs baseline: 2.0683x; 2.0683x over previous
import functools

import jax
import jax.numpy as jnp
from jax import lax
from jax.experimental import pallas as pl
from jax.experimental.pallas import tpu as pltpu
from jax.experimental.pallas import tpu_sc as plsc

_MAX_DIST = 1000
_NSHIFT = 16
_NUM_CORES = 2
_NUM_SUBCORES = 16


@functools.lru_cache(maxsize=None)
def _build_sc_kernel(S, H, VLEN):
    NW = _NUM_CORES * _NUM_SUBCORES
    ROWS = H * S
    RPW = ROWS // NW
    assert ROWS % NW == 0 and S % RPW == 0, (S, H)
    W = 2 * S
    assert VLEN == W + _NSHIFT
    CH = 16

    mesh = plsc.VectorSubcoreMesh(
        core_axis_name="c", subcore_axis_name="s",
        num_cores=_NUM_CORES, num_subcores=_NUM_SUBCORES)

    @functools.partial(
        pl.kernel,
        out_type=jax.ShapeDtypeStruct((ROWS * S,), jnp.float32),
        mesh=mesh,
        scratch_types=(
            [pltpu.VMEM((VLEN,), jnp.float32)]
            + [pltpu.VMEM((W,), jnp.float32)] * _NSHIFT
            + [pltpu.SemaphoreType.DMA]
        ),
    )
    def sc_kernel(v_hbm, out_hbm, vsrc, *rest):
        vs = rest[:_NSHIFT]
        sem = rest[_NSHIFT]
        wid = lax.axis_index("s") * _NUM_CORES + lax.axis_index("c")
        r0 = wid * RPW
        h = r0 // S
        i0 = r0 - h * S

        pltpu.sync_copy(v_hbm.at[h], vsrc)

        for m in range(_NSHIFT):
            def shift_body(kc, _, m=m):
                vs[m][pl.ds(kc * CH, CH)] = vsrc[pl.ds(m + kc * CH, CH)]
                return _
            lax.fori_loop(0, W // CH, shift_body, None)

        for m in range(_NSHIFT):
            o = (S - 1 - m) % _NSHIFT

            def fire(t, _, m=m, o=o):
                i = i0 + o + t * _NSHIFT
                a = pl.multiple_of((S - 1) - i - m, _NSHIFT)
                half = S // 2
                pltpu.make_async_copy(
                    vs[m].at[pl.ds(a, half)],
                    out_hbm.at[pl.ds((r0 + o + t * _NSHIFT) * S, half)],
                    sem).start()
                a2 = pl.multiple_of(a + half, _NSHIFT)
                pltpu.make_async_copy(
                    vs[m].at[pl.ds(a2, half)],
                    out_hbm.at[pl.ds((r0 + o + t * _NSHIFT) * S + half, half)],
                    sem).start()
                return _
            lax.fori_loop(0, RPW // _NSHIFT, fire, None)

        def drain(t, _):
            pltpu.make_async_copy(
                vs[0].at[pl.ds(0, S // 2)], out_hbm.at[pl.ds(r0 * S, S // 2)],
                sem).wait()
            return _
        lax.fori_loop(0, 2 * RPW, drain, None)

    return sc_kernel


def kernel(x, bias_values):
    S = x.shape[1]
    NB, H = bias_values.shape
    assert NB == 2 * _MAX_DIST
    VLEN = 2 * S + _NSHIFT
    n_left = (S - 1) - _MAX_DIST
    n_right = VLEN - n_left - NB
    assert n_left >= 0 and n_right >= 1

    v = jnp.concatenate([
        jnp.broadcast_to(bias_values[0], (n_left, H)),
        bias_values,
        jnp.broadcast_to(bias_values[-1], (n_right, H)),
    ], axis=0).T

    out = _build_sc_kernel(S, H, VLEN)(v)
    return out.reshape(1, H, S, S)


# speedup vs baseline: 2.1142x; 1.0222x over previous
import functools

import jax
import jax.numpy as jnp
from jax import lax
from jax.experimental import pallas as pl
from jax.experimental.pallas import tpu as pltpu
from jax.experimental.pallas import tpu_sc as plsc

_MAX_DIST = 1000
_NSHIFT = 16
_NUM_CORES = 2
_NUM_SUBCORES = 16


@functools.lru_cache(maxsize=None)
def _build_sc_kernel(S, H, VLEN):
    NW = _NUM_CORES * _NUM_SUBCORES
    ROWS = H * S
    RPW = ROWS // NW
    assert ROWS % NW == 0 and S % RPW == 0, (S, H)
    W = 2 * S
    assert VLEN == W + _NSHIFT
    CH = 16

    mesh = plsc.VectorSubcoreMesh(
        core_axis_name="c", subcore_axis_name="s",
        num_cores=_NUM_CORES, num_subcores=_NUM_SUBCORES)

    @functools.partial(
        pl.kernel,
        out_type=jax.ShapeDtypeStruct((ROWS * S,), jnp.float32),
        mesh=mesh,
        scratch_types=(
            [pltpu.VMEM((VLEN,), jnp.float32)]
            + [pltpu.VMEM((W,), jnp.float32)] * _NSHIFT
            + [pltpu.SemaphoreType.DMA] * 4
        ),
    )
    def sc_kernel(v_hbm, out_hbm, vsrc, *rest):
        vs = rest[:_NSHIFT]
        sems = rest[_NSHIFT:_NSHIFT + 4]
        wid = lax.axis_index("s") * _NUM_CORES + lax.axis_index("c")
        r0 = wid * RPW
        h = r0 // S
        i0 = r0 - h * S

        pltpu.sync_copy(v_hbm.at[h], vsrc)

        for m in range(_NSHIFT):
            def shift_body(kc, _, m=m):
                vs[m][pl.ds(kc * CH, CH)] = vsrc[pl.ds(m + kc * CH, CH)]
                return _
            lax.fori_loop(0, W // CH, shift_body, None)

        for m in range(_NSHIFT):
            o = (S - 1 - m) % _NSHIFT

            def fire(t, _, m=m, o=o):
                i = i0 + o + t * _NSHIFT
                a = pl.multiple_of((S - 1) - i - m, _NSHIFT)
                pltpu.make_async_copy(
                    vs[m].at[pl.ds(a, S)],
                    out_hbm.at[pl.ds((r0 + o + t * _NSHIFT) * S, S)],
                    sems[m % 4]).start()
                return _
            lax.fori_loop(0, RPW // _NSHIFT, fire, None)

        for q in range(4):
            def drain(t, _, q=q):
                pltpu.make_async_copy(
                    vs[0].at[pl.ds(0, S)], out_hbm.at[pl.ds(r0 * S, S)],
                    sems[q]).wait()
                return _
            lax.fori_loop(0, RPW // 4, drain, None)

    return sc_kernel


def kernel(x, bias_values):
    S = x.shape[1]
    NB, H = bias_values.shape
    assert NB == 2 * _MAX_DIST
    VLEN = 2 * S + _NSHIFT
    n_left = (S - 1) - _MAX_DIST
    n_right = VLEN - n_left - NB
    assert n_left >= 0 and n_right >= 1

    v = jnp.concatenate([
        jnp.broadcast_to(bias_values[0], (n_left, H)),
        bias_values,
        jnp.broadcast_to(bias_values[-1], (n_right, H)),
    ], axis=0).T

    out = _build_sc_kernel(S, H, VLEN)(v)
    return out.reshape(1, H, S, S)


# speedup vs baseline: 7.0551x; 3.3370x over previous
import functools

import jax
import jax.numpy as jnp
from jax import lax
from jax.experimental import pallas as pl
from jax.experimental.pallas import tpu as pltpu
from jax.experimental.pallas import tpu_sc as plsc

_MAX_DIST = 1000
_NSHIFT = 16
_NUM_CORES = 2
_NUM_SUBCORES = 16


@functools.lru_cache(maxsize=None)
def _build_sc_kernel(S, H, VLEN):
    NW = _NUM_CORES * _NUM_SUBCORES
    ROWS = H * S
    RPW = ROWS // NW
    assert ROWS % NW == 0 and S % RPW == 0, (S, H)
    W = 2 * S
    assert VLEN >= W + _NSHIFT
    CH = 16

    mesh = plsc.VectorSubcoreMesh(
        core_axis_name="c", subcore_axis_name="s",
        num_cores=_NUM_CORES, num_subcores=_NUM_SUBCORES)

    @functools.partial(
        pl.kernel,
        out_type=jax.ShapeDtypeStruct((ROWS * S,), jnp.float32),
        mesh=mesh,
        scratch_types=(
            [pltpu.VMEM((VLEN,), jnp.float32)]
            + [pltpu.VMEM((W,), jnp.float32)] * _NSHIFT
            + [pltpu.SemaphoreType.DMA] * 4
        ),
    )
    def sc_kernel(v_hbm, out_hbm, vsrc, *rest):
        vs = rest[:_NSHIFT]
        sems = rest[_NSHIFT:_NSHIFT + 4]
        wid = lax.axis_index("s") * _NUM_CORES + lax.axis_index("c")
        r0 = wid * RPW
        h = r0 // S
        i0 = r0 - h * S

        pltpu.sync_copy(v_hbm.at[h], vsrc)

        for m in range(_NSHIFT):
            def shift_body(kc, _, m=m):
                vs[m][pl.ds(kc * CH, CH)] = vsrc[pl.ds(m + kc * CH, CH)]
                return _
            lax.fori_loop(0, W // CH, shift_body, None)

        for m in range(_NSHIFT):
            o = (S - 1 - m) % _NSHIFT

            def fire(t, _, m=m, o=o):
                i = i0 + o + t * _NSHIFT
                a = pl.multiple_of((S - 1) - i - m, _NSHIFT)
                pltpu.make_async_copy(
                    vs[m].at[pl.ds(a, S)],
                    out_hbm.at[pl.ds((r0 + o + t * _NSHIFT) * S, S)],
                    sems[m % 4]).start()
                return _
            lax.fori_loop(0, RPW // _NSHIFT, fire, None)

        for q in range(4):
            def drain(t, _, q=q):
                pltpu.make_async_copy(
                    vs[0].at[pl.ds(0, S)], out_hbm.at[pl.ds(r0 * S, S)],
                    sems[q]).wait()
                return _
            lax.fori_loop(0, RPW // 4, drain, None)

    return sc_kernel


@functools.lru_cache(maxsize=None)
def _build_tc_kernel(S, H, VLEN):
    NP = 128
    RB = 256
    WV = 2 * S
    assert S % RB == 0 and RB % 128 == 0
    assert VLEN >= WV + NP

    def body(v_ref, out_ref, varrev):
        b = pl.program_id(1)

        @pl.when(b == 0)
        def _build():
            for p in range(NP):
                src_off = 8 * (p // 8) + 7 - (p % 8)
                varrev[p, :] = v_ref[0, 0, pl.ds(src_off, WV)]

        for rg in range(RB // 8):
            m = (S - 8 - 8 * rg) % NP
            a128 = pl.multiple_of(
                (S - 8 - 8 * rg - m) - b * RB, NP)
            out_ref[0, pl.ds(rg * 8, 8), :] = varrev[
                pl.ds(m, 8), pl.ds(a128, S)]

    return pl.pallas_call(
        body,
        grid=(H, S // RB),
        in_specs=[pl.BlockSpec((1, 1, VLEN), lambda h, b: (h, 0, 0))],
        out_specs=pl.BlockSpec((1, RB, S), lambda h, b: (h, b, 0)),
        out_shape=jax.ShapeDtypeStruct((H, S, S), jnp.float32),
        scratch_shapes=[pltpu.VMEM((NP, WV), jnp.float32)],
    )


_H_TC = 16


def kernel(x, bias_values):
    S = x.shape[1]
    NB, H = bias_values.shape
    assert NB == 2 * _MAX_DIST
    VLEN = 2 * S + 128
    n_left = (S - 1) - _MAX_DIST
    n_right = VLEN - n_left - NB
    assert n_left >= 0 and n_right >= 1

    v = jnp.concatenate([
        jnp.broadcast_to(bias_values[0], (n_left, H)),
        bias_values,
        jnp.broadcast_to(bias_values[-1], (n_right, H)),
    ], axis=0).T

    parts = []
    if _H_TC:
        tc = _build_tc_kernel(S, _H_TC, VLEN)(
            v[:_H_TC].reshape(_H_TC, 1, VLEN))
        parts.append(tc.reshape(1, _H_TC, S, S))
    h_sc = H - _H_TC
    if h_sc:
        sc = _build_sc_kernel(S, h_sc, VLEN)(v[_H_TC:])
        parts.append(sc.reshape(1, h_sc, S, S))
    return parts[0] if len(parts) == 1 else jnp.concatenate(parts, axis=1)
